# Initial kernel scaffold; baseline (speedup 1.0000x reference)
import jax
import jax.numpy as jnp
from jax.experimental import pallas as pl


def kernel(x, edge_index, batch, params):
    raise NotImplementedError("write your pallas kernel here")


# speedup vs baseline: 5.2970x; 5.2970x over previous
import functools

import jax
import jax.numpy as jnp
from jax import lax
from jax.experimental import pallas as pl
from jax.experimental.pallas import tpu as pltpu
from jax.experimental.pallas import tpu_sc as plsc

N = 10000
E = 320000
D = 128
H = 256
G = 16

NC = 2
NS = 16
NW = NC * NS
CH = 128
NCHUNK = E // CH
NP = 10240
RPS = NP // NS

@functools.lru_cache(maxsize=None)
def _build_segment_sum_sc():
    mesh = plsc.VectorSubcoreMesh(
        core_axis_name="c", subcore_axis_name="s",
        num_cores=NC, num_subcores=NS)

    @functools.partial(
        pl.kernel,
        out_type=jax.ShapeDtypeStruct((NC, NP, D), jnp.float32),
        mesh=mesh,
        scratch_types=[
            pltpu.VMEM((CH,), jnp.int32),
            pltpu.VMEM((CH,), jnp.int32),
            pltpu.VMEM((CH, D), jnp.float32),
            pltpu.VMEM_SHARED((NP, D), jnp.float32),
            pltpu.SemaphoreType.DMA,
        ],
    )
    def _segment_sum_sc(x_hbm, src_hbm, dst_hbm, zero_hbm, out_hbm,
                        src_v, dst_v, rows_v, acc_sh, sem):
        c = lax.axis_index("c")
        s = lax.axis_index("s")
        w = s * NC + c

        pltpu.sync_copy(zero_hbm, acc_sh.at[pl.ds(s * RPS, RPS)])
        plsc.subcore_barrier()

        n_extra = NCHUNK % NW
        nk = NCHUNK // NW + jnp.where(w < n_extra, 1, 0)

        def step(k, carry):
            base = (k * NW + w) * CH
            pltpu.sync_copy(src_hbm.at[pl.ds(base, CH)], src_v)
            pltpu.sync_copy(dst_hbm.at[pl.ds(base, CH)], dst_v)
            pltpu.async_copy(x_hbm.at[src_v], rows_v, sem).wait()
            pltpu.sync_copy(rows_v, acc_sh.at[dst_v], add=True)
            return carry

        lax.fori_loop(0, nk, step, 0)
        plsc.subcore_barrier()
        pltpu.sync_copy(acc_sh.at[pl.ds(s * RPS, RPS)],
                        out_hbm.at[c, pl.ds(s * RPS, RPS)])

    return _segment_sum_sc


def _bn_cols(t, g, b):
    mu = jnp.mean(t, axis=0, keepdims=True)
    var = jnp.mean((t - mu) ** 2, axis=0, keepdims=True)
    return (t - mu) / jnp.sqrt(var + 1e-5) * g + b


def _gin_mid_body(eps_ref, x_ref, a0_ref, a1_ref, w1_ref, g1_ref, b1_ref,
                  w2_ref, g2_ref, b2_ref, mask_ref, out_ref):
    h = (1.0 + eps_ref[0, 0]) * x_ref[...] + a0_ref[:N] + a1_ref[:N]
    t = jnp.dot(h, w1_ref[...], preferred_element_type=jnp.float32)
    t = _bn_cols(t, g1_ref[...], b1_ref[...])
    s = jnp.maximum(t, 0.0)
    u = jnp.dot(s, w2_ref[...], preferred_element_type=jnp.float32)
    u = _bn_cols(u, g2_ref[...], b2_ref[...])
    out_ref[...] = jnp.maximum(u * mask_ref[...], 0.0)


def _gin_last_body(eps_ref, x_ref, a0_ref, a1_ref, w1_ref, g1_ref, b1_ref,
                   w2_ref, mask_ref, batch_ref, out_ref):
    h = (1.0 + eps_ref[0, 0]) * x_ref[...] + a0_ref[:N] + a1_ref[:N]
    t = jnp.dot(h, w1_ref[...], preferred_element_type=jnp.float32)
    t = _bn_cols(t, g1_ref[...], b1_ref[...])
    s = jnp.maximum(t, 0.0)
    u = jnp.dot(s, w2_ref[...], preferred_element_type=jnp.float32)
    xf = jnp.maximum(u * mask_ref[...], 0.0)
    oh = (lax.broadcasted_iota(jnp.int32, (G, N), 0)
          == batch_ref[...]).astype(jnp.float32)
    sums = jnp.dot(oh, xf, preferred_element_type=jnp.float32)
    cnt = jnp.sum(oh, axis=1, keepdims=True)
    out_ref[...] = sums / jnp.maximum(cnt, 1.0)


_tc_params = pltpu.CompilerParams(vmem_limit_bytes=100 * 1024 * 1024)

_mid_call = pl.pallas_call(
    _gin_mid_body,
    out_shape=jax.ShapeDtypeStruct((N, D), jnp.float32),
    in_specs=[pl.BlockSpec(memory_space=pltpu.SMEM)]
    + [pl.BlockSpec(memory_space=pltpu.VMEM)] * 10,
    out_specs=pl.BlockSpec(memory_space=pltpu.VMEM),
    compiler_params=_tc_params,
)

_last_call = pl.pallas_call(
    _gin_last_body,
    out_shape=jax.ShapeDtypeStruct((G, D), jnp.float32),
    in_specs=[pl.BlockSpec(memory_space=pltpu.SMEM)]
    + [pl.BlockSpec(memory_space=pltpu.VMEM)] * 9,
    out_specs=pl.BlockSpec(memory_space=pltpu.VMEM),
    compiler_params=_tc_params,
)


def _make_mask(i):
    od = D if i < 6 else 2
    dk = jax.random.key(42)
    keep = jax.random.bernoulli(jax.random.fold_in(dk, i), 0.8, (N, od))
    m = jnp.where(keep, jnp.float32(1.0) / jnp.float32(0.8), jnp.float32(0.0))
    if od != D:
        m = jnp.pad(m, ((0, 0), (0, D - od)))
    return m


def kernel(x, edge_index, batch, params):
    src = edge_index[0]
    dst = edge_index[1]
    zeros = jnp.zeros((RPS, D), jnp.float32)
    batch_row = batch.reshape(1, N)
    out = None
    seg_sum = _build_segment_sum_sc()
    for i, p in enumerate(params):
        parts = seg_sum(x, src, dst, zeros)
        eps2 = p['eps'].reshape(1, 1)
        if i < 6:
            x = _mid_call(eps2, x, parts[0], parts[1], p['W1'],
                          p['g1'].reshape(1, H), p['b1'].reshape(1, H),
                          p['W2'], p['g2'].reshape(1, D),
                          p['b2'].reshape(1, D), _make_mask(i))
        else:
            w2p = jnp.pad(p['W2'], ((0, 0), (0, D - 2)))
            out = _last_call(eps2, x, parts[0], parts[1], p['W1'],
                             p['g1'].reshape(1, H), p['b1'].reshape(1, H),
                             w2p, _make_mask(6), batch_row)
    return out[:, :2]


# speedup vs baseline: 9.4510x; 1.7842x over previous
import functools

import jax
import jax.numpy as jnp
from jax import lax
from jax.experimental import pallas as pl
from jax.experimental.pallas import tpu as pltpu
from jax.experimental.pallas import tpu_sc as plsc

N = 10000
E = 320000
D = 128
H = 256
G = 16

NC = 2
NS = 16
NW = NC * NS
CH = 128
NK = 80
NCHUNKP = NW * NK
EPAD = NCHUNKP * CH
NP = 10240
RPS = NP // NS

@functools.lru_cache(maxsize=None)
def _build_segment_sum_sc():
    mesh = plsc.VectorSubcoreMesh(
        core_axis_name="c", subcore_axis_name="s",
        num_cores=NC, num_subcores=NS)

    @functools.partial(
        pl.kernel,
        out_type=jax.ShapeDtypeStruct((NC, NP, D), jnp.float32),
        mesh=mesh,
        scratch_types=[
            pltpu.VMEM((2, CH), jnp.int32),
            pltpu.VMEM((2, CH), jnp.int32),
            pltpu.VMEM((CH, D), jnp.float32),
            pltpu.VMEM((CH, D), jnp.float32),
            pltpu.VMEM_SHARED((NP, D), jnp.float32),
            pltpu.SemaphoreType.DMA,
            pltpu.SemaphoreType.DMA,
        ],
    )
    def _segment_sum_sc(x_hbm, idx_hbm, zero_hbm, out_hbm,
                        idx0, idx1, rows0, rows1, acc_sh, gs0, gs1):
        c = lax.axis_index("c")
        s = lax.axis_index("s")
        w = s * NC + c

        pltpu.sync_copy(zero_hbm.at[pl.ds(s * RPS, RPS)],
                        acc_sh.at[pl.ds(s * RPS, RPS)])
        plsc.subcore_barrier()

        base = w * NK
        pltpu.sync_copy(idx_hbm.at[base], idx0)
        pltpu.async_copy(x_hbm.at[idx0.at[0]], rows0, gs0)
        pltpu.sync_copy(idx_hbm.at[base + 1], idx1)
        pltpu.async_copy(x_hbm.at[idx1.at[0]], rows1, gs1)

        def step(j, carry):
            k0 = 2 * j

            pltpu.make_async_copy(x_hbm.at[idx0.at[0]], rows0, gs0).wait()
            pltpu.sync_copy(rows0, acc_sh.at[idx0.at[1]], add=True)

            @pl.when(k0 + 2 < NK)
            def _():
                pltpu.sync_copy(idx_hbm.at[base + k0 + 2], idx0)
                pltpu.async_copy(x_hbm.at[idx0.at[0]], rows0, gs0)

            pltpu.make_async_copy(x_hbm.at[idx1.at[0]], rows1, gs1).wait()
            pltpu.sync_copy(rows1, acc_sh.at[idx1.at[1]], add=True)

            @pl.when(k0 + 3 < NK)
            def _():
                pltpu.sync_copy(idx_hbm.at[base + k0 + 3], idx1)
                pltpu.async_copy(x_hbm.at[idx1.at[0]], rows1, gs1)

            return carry

        lax.fori_loop(0, NK // 2, step, 0)
        plsc.subcore_barrier()
        pltpu.sync_copy(acc_sh.at[pl.ds(s * RPS, RPS)],
                        out_hbm.at[c, pl.ds(s * RPS, RPS)])

    return _segment_sum_sc


def _bn_cols(t, g, b):
    mu = jnp.mean(t, axis=0, keepdims=True)
    var = jnp.mean((t - mu) ** 2, axis=0, keepdims=True)
    return (t - mu) / jnp.sqrt(var + 1e-5) * g + b


def _gin_mid_body(eps_ref, x_ref, a0_ref, a1_ref, w1_ref, g1_ref, b1_ref,
                  w2_ref, g2_ref, b2_ref, mask_ref, out_ref):
    h = (1.0 + eps_ref[0, 0]) * x_ref[...] + a0_ref[:N] + a1_ref[:N]
    t = jnp.dot(h, w1_ref[...], preferred_element_type=jnp.float32)
    t = _bn_cols(t, g1_ref[...], b1_ref[...])
    s = jnp.maximum(t, 0.0)
    u = jnp.dot(s, w2_ref[...], preferred_element_type=jnp.float32)
    u = _bn_cols(u, g2_ref[...], b2_ref[...])
    out_ref[...] = jnp.maximum(u * mask_ref[...], 0.0)


def _gin_last_body(eps_ref, x_ref, a0_ref, a1_ref, w1_ref, g1_ref, b1_ref,
                   w2_ref, mask_ref, batch_ref, out_ref):
    h = (1.0 + eps_ref[0, 0]) * x_ref[...] + a0_ref[:N] + a1_ref[:N]
    t = jnp.dot(h, w1_ref[...], preferred_element_type=jnp.float32)
    t = _bn_cols(t, g1_ref[...], b1_ref[...])
    s = jnp.maximum(t, 0.0)
    u = jnp.dot(s, w2_ref[...], preferred_element_type=jnp.float32)
    xf = jnp.maximum(u * mask_ref[...], 0.0)
    oh = (lax.broadcasted_iota(jnp.int32, (G, N), 0)
          == batch_ref[...]).astype(jnp.float32)
    sums = jnp.dot(oh, xf, preferred_element_type=jnp.float32)
    cnt = jnp.sum(oh, axis=1, keepdims=True)
    out_ref[...] = sums / jnp.maximum(cnt, 1.0)


_tc_params = pltpu.CompilerParams(vmem_limit_bytes=100 * 1024 * 1024)

_mid_call = pl.pallas_call(
    _gin_mid_body,
    out_shape=jax.ShapeDtypeStruct((N, D), jnp.float32),
    in_specs=[pl.BlockSpec(memory_space=pltpu.SMEM)]
    + [pl.BlockSpec(memory_space=pltpu.VMEM)] * 10,
    out_specs=pl.BlockSpec(memory_space=pltpu.VMEM),
    compiler_params=_tc_params,
)

_last_call = pl.pallas_call(
    _gin_last_body,
    out_shape=jax.ShapeDtypeStruct((G, D), jnp.float32),
    in_specs=[pl.BlockSpec(memory_space=pltpu.SMEM)]
    + [pl.BlockSpec(memory_space=pltpu.VMEM)] * 9,
    out_specs=pl.BlockSpec(memory_space=pltpu.VMEM),
    compiler_params=_tc_params,
)


def _make_mask(i):
    od = D if i < 6 else 2
    dk = jax.random.key(42)
    keep = jax.random.bernoulli(jax.random.fold_in(dk, i), 0.8, (N, od))
    m = jnp.where(keep, jnp.float32(1.0) / jnp.float32(0.8), jnp.float32(0.0))
    if od != D:
        m = jnp.pad(m, ((0, 0), (0, D - od)))
    return m


def kernel(x, edge_index, batch, params):
    npad = EPAD - E
    pad_iota = jnp.arange(npad, dtype=jnp.int32)
    srcp = jnp.concatenate([edge_index[0], pad_iota % N])
    dstp = jnp.concatenate([edge_index[1], N + pad_iota % (NP - N)])
    idx = jnp.stack([srcp, dstp]).reshape(2, NCHUNKP, CH).transpose(1, 0, 2)
    zeros = jnp.zeros((NP, D), jnp.float32)
    batch_row = batch.reshape(1, N)
    out = None
    seg_sum = _build_segment_sum_sc()
    for i, p in enumerate(params):
        parts = seg_sum(x, idx, zeros)
        eps2 = p['eps'].reshape(1, 1)
        if i < 6:
            x = _mid_call(eps2, x, parts[0], parts[1], p['W1'],
                          p['g1'].reshape(1, H), p['b1'].reshape(1, H),
                          p['W2'], p['g2'].reshape(1, D),
                          p['b2'].reshape(1, D), _make_mask(i))
        else:
            w2p = jnp.pad(p['W2'], ((0, 0), (0, D - 2)))
            out = _last_call(eps2, x, parts[0], parts[1], p['W1'],
                             p['g1'].reshape(1, H), p['b1'].reshape(1, H),
                             w2p, _make_mask(6), batch_row)
    return out[:, :2]
